# baseline (speedup 1.0000x reference)
import functools
import math

import jax
import jax.numpy as jnp
from jax import lax
from jax.experimental import pallas as pl
from jax.experimental.pallas import tpu as pltpu
from jax.experimental.pallas import tpu_sc as plsc

D_MODEL = 64
SCALE = math.sqrt(D_MODEL)

NC = 2
NS = 16
NW = NC * NS

LANES = 128
NBUF = 4


def _sc_embed(xt, lut2, n_t, n_c):
    tasks = n_t * n_c
    tpw = tasks // NW
    mesh = plsc.VectorSubcoreMesh(core_axis_name="c", subcore_axis_name="s")

    @functools.partial(
        pl.kernel,
        mesh=mesh,
        out_type=jax.ShapeDtypeStruct((n_t, 8, n_c, 8, LANES), jnp.float32),
        scratch_types=[
            pltpu.VMEM((tpw, LANES), jnp.int32),
            pltpu.VMEM((NBUF, LANES, 2 * D_MODEL), jnp.float32),
            pltpu.VMEM((NBUF, 8, 8, LANES + 1), jnp.float32),
            pltpu.SemaphoreType.DMA((NBUF,)),
            pltpu.SemaphoreType.DMA((NBUF,)),
        ],
        compiler_params=pltpu.CompilerParams(
            use_tc_tiling_on_sc=False, needs_layout_passes=False
        ),
    )
    def k(xt_hbm, lut_hbm, out_hbm, idx_all, inbuf, tilebuf, gsem, ssem):
        wid = lax.axis_index("s") * NC + lax.axis_index("c")
        k0 = wid * tpw

        pltpu.sync_copy(xt_hbm.at[pl.ds(k0, tpw)], idx_all)

        def fire_gather(kk, b):
            pltpu.async_copy(lut_hbm.at[idx_all.at[kk]], inbuf.at[b], gsem.at[b])

        def wait_gather(kk, b):
            pltpu.make_async_copy(
                lut_hbm.at[idx_all.at[kk]], inbuf.at[b], gsem.at[b]
            ).wait()

        def fire_store(kk, b):
            tau = k0 + kk
            t = tau // n_c
            c = tau % n_c
            pltpu.async_copy(
                tilebuf.at[b, :, :, pl.ds(0, LANES)],
                out_hbm.at[t, :, c],
                ssem.at[b],
            )

        def wait_store(kk, b):
            tau = k0 + kk
            t = tau // n_c
            c = tau % n_c
            pltpu.make_async_copy(
                tilebuf.at[b, :, :, pl.ds(0, LANES)],
                out_hbm.at[t, :, c],
                ssem.at[b],
            ).wait()

        dios = [lax.iota(jnp.int32, 16) + 16 * g for g in range(D_MODEL // 16)]
        rvecs = [lax.shift_right_logical(d, 3) for d in dios]
        svecs = [lax.bitwise_and(d, 7) for d in dios]
        zeros16 = jnp.zeros((16,), jnp.int32)

        for b in range(NBUF):
            fire_gather(b, b)

        def outer(jbase, carry):
            for b in range(NBUF):
                j = jbase + b
                wait_gather(j, b)

                @pl.when(j >= NBUF)
                def _():
                    wait_store(j - NBUF, b)

                @plsc.parallel_loop(0, LANES, unroll=4)
                def tr(row):
                    rowv = zeros16 + row
                    for q in range(D_MODEL // 16):
                        v = inbuf[b, row, pl.ds(16 * q, 16)] * SCALE
                        plsc.store_scatter(
                            tilebuf.at[b], [rvecs[q], svecs[q], rowv], v
                        )

                @pl.when(j + NBUF < tpw)
                def _():
                    fire_gather(j + NBUF, b)

                fire_store(j, b)
            return carry

        lax.fori_loop(0, tpw // NBUF, lambda i, c: outer(i * NBUF, c), 0)

        for b in range(NBUF):
            wait_store(tpw - NBUF + b, b)

    return k(xt, lut2)


def kernel(x, lut):
    n_b, n_t = x.shape
    n_c = n_b // LANES
    xt = jnp.transpose(x).reshape(n_t * n_c, LANES).astype(jnp.int32)
    lut2 = jnp.pad(lut, ((0, 0), (0, D_MODEL)))
    out_p = _sc_embed(xt, lut2, n_t, n_c)
    return out_p.transpose(2, 4, 0, 1, 3).reshape(n_b, n_t, D_MODEL)


# speedup vs baseline: 1.0138x; 1.0138x over previous
import functools
import math

import jax
import jax.numpy as jnp
from jax import lax
from jax.experimental import pallas as pl
from jax.experimental.pallas import tpu as pltpu
from jax.experimental.pallas import tpu_sc as plsc

D_MODEL = 64
SCALE = math.sqrt(D_MODEL)

NC = 2
NS = 16
NW = NC * NS

LANES = 128
NBUF = 4


def _sc_embed(xt, lut, n_t, n_c):
    tasks = n_t * n_c
    tpw = tasks // NW
    mesh = plsc.VectorSubcoreMesh(core_axis_name="c", subcore_axis_name="s")

    @functools.partial(
        pl.kernel,
        mesh=mesh,
        out_type=jax.ShapeDtypeStruct((n_t, 8, n_c, 8, LANES), jnp.float32),
        scratch_types=[
            pltpu.VMEM((tpw, LANES), jnp.int32),
            pltpu.VMEM((NBUF, LANES, D_MODEL), jnp.float32),
            pltpu.VMEM((NBUF, 8, 8, LANES + 1), jnp.float32),
            pltpu.SemaphoreType.DMA((NBUF,)),
            pltpu.SemaphoreType.DMA((NBUF,)),
        ],
        compiler_params=pltpu.CompilerParams(
            use_tc_tiling_on_sc=False, needs_layout_passes=False
        ),
    )
    def k(xt_hbm, lut_hbm, out_hbm, idx_all, inbuf, tilebuf, gsem, ssem):
        wid = lax.axis_index("s") * NC + lax.axis_index("c")
        k0 = wid * tpw

        pltpu.sync_copy(xt_hbm.at[pl.ds(k0, tpw)], idx_all)

        def fire_gather(kk, b):
            pltpu.async_copy(lut_hbm.at[idx_all.at[kk]], inbuf.at[b], gsem.at[b])

        def wait_gather(kk, b):
            pltpu.make_async_copy(
                lut_hbm.at[idx_all.at[kk]], inbuf.at[b], gsem.at[b]
            ).wait()

        def fire_store(kk, b):
            tau = k0 + kk
            t = tau // n_c
            c = tau % n_c
            pltpu.async_copy(
                tilebuf.at[b, :, :, pl.ds(0, LANES)],
                out_hbm.at[t, :, c],
                ssem.at[b],
            )

        def wait_store(kk, b):
            tau = k0 + kk
            t = tau // n_c
            c = tau % n_c
            pltpu.make_async_copy(
                tilebuf.at[b, :, :, pl.ds(0, LANES)],
                out_hbm.at[t, :, c],
                ssem.at[b],
            ).wait()

        dios = [lax.iota(jnp.int32, 16) + 16 * g for g in range(D_MODEL // 16)]
        rvecs = [lax.shift_right_logical(d, 3) for d in dios]
        svecs = [lax.bitwise_and(d, 7) for d in dios]
        zeros16 = jnp.zeros((16,), jnp.int32)

        for b in range(NBUF):
            fire_gather(b, b)

        def outer(jbase, carry):
            for b in range(NBUF):
                j = jbase + b
                wait_gather(j, b)

                @pl.when(j >= NBUF)
                def _():
                    wait_store(j - NBUF, b)

                @plsc.parallel_loop(0, LANES, unroll=4)
                def tr(row):
                    rowv = zeros16 + row
                    for q in range(D_MODEL // 16):
                        v = inbuf[b, row, pl.ds(16 * q, 16)] * SCALE
                        plsc.store_scatter(
                            tilebuf.at[b], [rvecs[q], svecs[q], rowv], v
                        )

                @pl.when(j + NBUF < tpw)
                def _():
                    fire_gather(j + NBUF, b)

                fire_store(j, b)
            return carry

        lax.fori_loop(0, tpw // NBUF, lambda i, c: outer(i * NBUF, c), 0)

        for b in range(NBUF):
            wait_store(tpw - NBUF + b, b)

    return k(xt, lut)


def kernel(x, lut):
    n_b, n_t = x.shape
    n_c = n_b // LANES
    xt = jnp.transpose(x).reshape(n_t * n_c, LANES).astype(jnp.int32)
    out_p = _sc_embed(xt, lut, n_t, n_c)
    return out_p.transpose(2, 4, 0, 1, 3).reshape(n_b, n_t, D_MODEL)
